# baseline (speedup 1.0000x reference)
import functools

import jax
import jax.numpy as jnp
from jax import lax
from jax.experimental import pallas as pl
from jax.experimental.pallas import tpu as pltpu
from jax.experimental.pallas import tpu_sc as plsc

C, T, H, W = 3, 64, 384, 384
TS = T // 4
PPF = 6
QROWS = H // PPF
NW = 32
PER_W = C * T * PPF // NW
NBUF = 5


def _sc_fast_copy(frames):
    mesh = plsc.VectorSubcoreMesh(core_axis_name="c", subcore_axis_name="s")

    @functools.partial(
        pl.kernel,
        mesh=mesh,
        out_type=jax.ShapeDtypeStruct((C, T, H, W), jnp.float32),
        scratch_types=[
            pltpu.VMEM((NBUF, QROWS, W), jnp.float32),
            pltpu.SemaphoreType.DMA,
            pltpu.SemaphoreType.DMA,
        ],
        compiler_params=pltpu.CompilerParams(use_tc_tiling_on_sc=True),
    )
    def k(src, fast_out, buf, sem_r, sem_w):
        wid = lax.axis_index("s") * 2 + lax.axis_index("c")

        def coords(j):
            p = wid * PER_W + j
            return p // (T * PPF), (p // PPF) % T, p % PPF

        def rd(j):
            c, t, q = coords(j)
            rows = pl.ds(q * QROWS, QROWS)
            return pltpu.make_async_copy(
                src.at[c, t, rows], buf.at[j % NBUF], sem_r
            )

        def wr(j):
            c, t, q = coords(j)
            rows = pl.ds(q * QROWS, QROWS)
            return pltpu.make_async_copy(
                buf.at[j % NBUF], fast_out.at[c, t, rows], sem_w
            )

        rd(0).start()
        rd(1).start()
        for j in range(PER_W):
            rd(j).wait()
            if j >= 3:
                wr(j - 3).wait()
            wr(j).start()
            if j + 2 < PER_W:
                rd(j + 2).start()
        for j in range(max(0, PER_W - 3), PER_W):
            wr(j).wait()

    return k(frames)


def _tc_slow_gather(frames):
    def body(src_ref, out_ref):
        out_ref[...] = src_ref[...]

    return pl.pallas_call(
        body,
        grid=(C, TS, 4),
        in_specs=[
            pl.BlockSpec(
                (1, 1, H // 4, W), lambda c, i, q: (c, (i * 21) // 5, q, 0)
            )
        ],
        out_specs=pl.BlockSpec((1, 1, H // 4, W), lambda c, i, q: (c, i, q, 0)),
        out_shape=jax.ShapeDtypeStruct((C, TS, H, W), jnp.float32),
    )(frames)


def kernel(frames):
    fast = _sc_fast_copy(frames)
    slow = _tc_slow_gather(frames)
    return (slow, fast)


# speedup vs baseline: 1.4714x; 1.4714x over previous
import functools

import jax
import jax.numpy as jnp
from jax import lax
from jax.experimental import pallas as pl
from jax.experimental.pallas import tpu as pltpu
from jax.experimental.pallas import tpu_sc as plsc

C, T, H, W = 3, 64, 384, 384
TS = T // 4
PPF = 6
QROWS = H // PPF
NW = 32
PER_W = C * T * PPF // NW
NBUF = 5


def _sc_fast_copy(frames):
    mesh = plsc.VectorSubcoreMesh(core_axis_name="c", subcore_axis_name="s")

    @functools.partial(
        pl.kernel,
        mesh=mesh,
        out_type=jax.ShapeDtypeStruct((C, T, H, W), jnp.float32),
        scratch_types=[
            pltpu.VMEM((NBUF, QROWS, W), jnp.float32),
            pltpu.SemaphoreType.DMA,
            pltpu.SemaphoreType.DMA,
        ],
        compiler_params=pltpu.CompilerParams(use_tc_tiling_on_sc=True),
    )
    def k(src, fast_out, buf, sem_r, sem_w):
        wid = lax.axis_index("s") * 2 + lax.axis_index("c")

        def coords(j):
            p = wid * PER_W + j
            return p // (T * PPF), (p // PPF) % T, p % PPF

        def rd(j):
            c, t, q = coords(j)
            rows = pl.ds(q * QROWS, QROWS)
            return pltpu.make_async_copy(
                src.at[c, t, rows], buf.at[j % NBUF], sem_r
            )

        def wr(j):
            c, t, q = coords(j)
            rows = pl.ds(q * QROWS, QROWS)
            return pltpu.make_async_copy(
                buf.at[j % NBUF], fast_out.at[c, t, rows], sem_w
            )

        rd(0).start()
        rd(1).start()
        for j in range(PER_W):
            rd(j).wait()
            if j >= 3:
                wr(j - 3).wait()
            wr(j).start()
            if j + 2 < PER_W:
                rd(j + 2).start()
        for j in range(max(0, PER_W - 3), PER_W):
            wr(j).wait()

    return k(frames)


def _tc_slow_gather(frames):
    def body(src_ref, out_ref):
        out_ref[...] = src_ref[...]

    return pl.pallas_call(
        body,
        grid=(C, TS),
        in_specs=[
            pl.BlockSpec((1, 1, H, W), lambda c, i: (c, (i * 21) // 5, 0, 0))
        ],
        out_specs=pl.BlockSpec((1, 1, H, W), lambda c, i: (c, i, 0, 0)),
        out_shape=jax.ShapeDtypeStruct((C, TS, H, W), jnp.float32),
    )(frames)


def kernel(frames):
    fast = _sc_fast_copy(frames)
    slow = _tc_slow_gather(frames)
    return (slow, fast)
